# Initial kernel scaffold; baseline (speedup 1.0000x reference)
import functools

import jax
import jax.numpy as jnp
from jax import lax
from jax.experimental import pallas as pl
from jax.experimental.pallas import tpu as pltpu
from jax.experimental.pallas import tpu_sc as plsc

NC = 2
NS = 16
CH = 128
LANES = 16


def _sc_scatter(x, src, dst, w, chunks_per_tile):
    n, d = x.shape
    per_tile = chunks_per_tile * CH
    rows_per_sub = n // NS
    mesh = plsc.VectorSubcoreMesh(core_axis_name="c", subcore_axis_name="s")

    @functools.partial(
        pl.kernel,
        out_type=jax.ShapeDtypeStruct((NC, n, d), jnp.float32),
        mesh=mesh,
        scratch_types=[
            pltpu.VMEM((CH,), jnp.int32),
            pltpu.VMEM((CH,), jnp.int32),
            pltpu.VMEM((CH,), jnp.float32),
            pltpu.VMEM((CH, d), jnp.float32),
            pltpu.VMEM_SHARED((n, d), jnp.float32),
            pltpu.SemaphoreType.DMA,
        ],
    )
    def scat(x_hbm, src_hbm, dst_hbm, w_hbm, out_hbm,
             src_v, dst_v, w_v, rows_v, acc_sh, sem):
        cid = lax.axis_index("c")
        sid = lax.axis_index("s")

        @pl.loop(0, CH)
        def _(i):
            for j in range(d // LANES):
                rows_v[i, pl.ds(j * LANES, LANES)] = jnp.zeros((LANES,), jnp.float32)

        base_row = sid * rows_per_sub
        done = 0
        while done < rows_per_sub:
            step = min(CH, rows_per_sub - done)
            pltpu.sync_copy(rows_v.at[pl.ds(0, step)],
                            acc_sh.at[pl.ds(base_row + done, step)])
            done += step
        plsc.subcore_barrier()

        base_e = (cid * NS + sid) * per_tile

        @pl.loop(0, chunks_per_tile)
        def _(ci):
            off = base_e + ci * CH
            pltpu.sync_copy(src_hbm.at[pl.ds(off, CH)], src_v)
            pltpu.sync_copy(dst_hbm.at[pl.ds(off, CH)], dst_v)
            pltpu.sync_copy(w_hbm.at[pl.ds(off, CH)], w_v)
            pltpu.async_copy(x_hbm.at[src_v], rows_v, sem).wait()

            @pl.loop(0, CH)
            def _(i):
                bidx = jnp.full((LANES,), i, jnp.int32)
                wb = plsc.load_gather(w_v, [bidx])
                for j in range(d // LANES):
                    sl = pl.ds(j * LANES, LANES)
                    rows_v[i, sl] = rows_v[i, sl] * wb

            pltpu.sync_copy(rows_v, acc_sh.at[dst_v], add=True)

        plsc.subcore_barrier()
        pltpu.sync_copy(acc_sh.at[pl.ds(base_row, rows_per_sub)],
                        out_hbm.at[cid, pl.ds(base_row, rows_per_sub)])

    return scat(x, src, dst, w)


def _mm_body(y0_ref, y1_ref, w_ref, o_ref):
    s = y0_ref[...] + y1_ref[...]
    o_ref[...] = jnp.dot(s, w_ref[...], preferred_element_type=jnp.float32)


def _combine_matmul(y0, y1, W, blk):
    n, d_in = y0.shape
    d_out = W.shape[1]
    return pl.pallas_call(
        _mm_body,
        grid=(n // blk,),
        in_specs=[
            pl.BlockSpec((blk, d_in), lambda i: (i, 0)),
            pl.BlockSpec((blk, d_in), lambda i: (i, 0)),
            pl.BlockSpec((d_in, d_out), lambda i: (0, 0)),
        ],
        out_specs=pl.BlockSpec((blk, d_out), lambda i: (i, 0)),
        out_shape=jax.ShapeDtypeStruct((n, d_out), jnp.float32),
    )(y0, y1, W)


def kernel(x, edge_index, edge_weight, W):
    n, _ = x.shape
    e = edge_index.shape[1]
    quota = NC * NS * CH
    ep = ((e + quota - 1) // quota) * quota
    pad = ep - e

    src = edge_index[1].astype(jnp.int32)
    dst = edge_index[0].astype(jnp.int32)
    w = edge_weight
    if pad:
        src = jnp.concatenate([src, jnp.zeros((pad,), jnp.int32)])
        dst = jnp.concatenate([dst, jnp.zeros((pad,), jnp.int32)])
        w = jnp.concatenate([w, jnp.zeros((pad,), jnp.float32)])

    y = _sc_scatter(x, src, dst, w, ep // quota)
    return _combine_matmul(y[0], y[1], W, blk=400)


# speedup vs baseline: 3.0318x; 3.0318x over previous
import dataclasses
import functools

import jax
import jax.numpy as jnp
from jax import lax
from jax.experimental import pallas as pl
from jax.experimental.pallas import tpu as pltpu
from jax.experimental.pallas import tpu_sc as plsc

NC = 2
NS = 16
CH = 128
LANES = 16


def _sc_scatter(x, src, dst, w, chunks_per_tile):
    n, d = x.shape
    per_tile = chunks_per_tile * CH
    npad = ((n + 8 * NS - 1) // (8 * NS)) * (8 * NS)
    rows_per_sub = npad // NS
    mesh = plsc.VectorSubcoreMesh(core_axis_name="c", subcore_axis_name="s")
    cp = pltpu.CompilerParams()
    if "needs_layout_passes" in pltpu.CompilerParams.__dataclass_fields__:
        cp = dataclasses.replace(cp, needs_layout_passes=False)

    @functools.partial(
        pl.kernel,
        out_type=jax.ShapeDtypeStruct((NC, npad, d), jnp.float32),
        mesh=mesh,
        compiler_params=cp,
        scratch_types=[
            pltpu.VMEM((CH,), jnp.int32),
            pltpu.VMEM((CH,), jnp.int32),
            pltpu.VMEM((CH,), jnp.float32),
            pltpu.VMEM((CH, d), jnp.float32),
            pltpu.VMEM_SHARED((npad, d), jnp.float32),
            pltpu.SemaphoreType.DMA,
        ],
    )
    def scat(x_hbm, src_hbm, dst_hbm, w_hbm, out_hbm,
             src_v, dst_v, w_v, rows_v, acc_sh, sem):
        cid = lax.axis_index("c")
        sid = lax.axis_index("s")

        @pl.loop(0, CH)
        def _(i):
            for j in range(d // LANES):
                rows_v[i, pl.ds(j * LANES, LANES)] = jnp.zeros((LANES,), jnp.float32)

        base_row = sid * rows_per_sub
        done = 0
        while done < rows_per_sub:
            step = min(CH, rows_per_sub - done)
            pltpu.sync_copy(rows_v.at[pl.ds(0, step)],
                            acc_sh.at[pl.ds(base_row + done, step)])
            done += step
        plsc.subcore_barrier()

        base_e = (cid * NS + sid) * per_tile

        @pl.loop(0, chunks_per_tile)
        def _(ci):
            off = base_e + ci * CH
            pltpu.sync_copy(src_hbm.at[pl.ds(off, CH)], src_v)
            pltpu.sync_copy(dst_hbm.at[pl.ds(off, CH)], dst_v)
            pltpu.sync_copy(w_hbm.at[pl.ds(off, CH)], w_v)
            pltpu.async_copy(x_hbm.at[src_v], rows_v, sem).wait()

            @pl.loop(0, CH)
            def _(i):
                bidx = jnp.full((LANES,), i, jnp.int32)
                wb = plsc.load_gather(w_v, [bidx])
                for j in range(d // LANES):
                    sl = pl.ds(j * LANES, LANES)
                    rows_v[i, sl] = rows_v[i, sl] * wb

            pltpu.sync_copy(rows_v, acc_sh.at[dst_v], add=True)

        plsc.subcore_barrier()
        pltpu.sync_copy(acc_sh.at[pl.ds(base_row, rows_per_sub)],
                        out_hbm.at[cid, pl.ds(base_row, rows_per_sub)])

    return scat(x, src, dst, w)


def _mm_body(y0_ref, y1_ref, w_ref, o_ref):
    s = y0_ref[...] + y1_ref[...]
    o_ref[...] = jnp.dot(s, w_ref[...], preferred_element_type=jnp.float32)


def _combine_matmul(y0, y1, W, n, blk):
    d_in = y0.shape[1]
    d_out = W.shape[1]
    return pl.pallas_call(
        _mm_body,
        grid=(n // blk,),
        in_specs=[
            pl.BlockSpec((blk, d_in), lambda i: (i, 0)),
            pl.BlockSpec((blk, d_in), lambda i: (i, 0)),
            pl.BlockSpec((d_in, d_out), lambda i: (0, 0)),
        ],
        out_specs=pl.BlockSpec((blk, d_out), lambda i: (i, 0)),
        out_shape=jax.ShapeDtypeStruct((n, d_out), jnp.float32),
    )(y0, y1, W)


def kernel(x, edge_index, edge_weight, W):
    n, _ = x.shape
    e = edge_index.shape[1]
    quota = NC * NS * CH
    ep = ((e + quota - 1) // quota) * quota
    pad = ep - e

    src = edge_index[1].astype(jnp.int32)
    dst = edge_index[0].astype(jnp.int32)
    w = edge_weight
    if pad:
        src = jnp.concatenate([src, jnp.zeros((pad,), jnp.int32)])
        dst = jnp.concatenate([dst, jnp.zeros((pad,), jnp.int32)])
        w = jnp.concatenate([w, jnp.zeros((pad,), jnp.float32)])

    y = _sc_scatter(x, src, dst, w, ep // quota)
    return _combine_matmul(y[0], y[1], W, n, blk=400)
